# Initial kernel scaffold; baseline (speedup 1.0000x reference)
import jax
import jax.numpy as jnp
from jax.experimental import pallas as pl


def kernel(x, edge_index, batch, W1, b1, W2, b2, Wfc, bfc):
    raise NotImplementedError("write your pallas kernel here")


# speedup vs baseline: 14.7372x; 14.7372x over previous
import functools

import jax
import jax.numpy as jnp
from jax import lax
from jax.experimental import pallas as pl
from jax.experimental.pallas import tpu as pltpu
from jax.experimental.pallas import tpu_sc as plsc

N_NODES = 10000
N_EDGES = 320000
D_FEAT = 128
HIDDEN = 256
ACTION_DIM = 64
NUM_GRAPHS = 64

NC = 2
NS = 16
NW = NC * NS
E_PER_W = N_EDGES // NW
K = 128
NFULL = E_PER_W // K
KTAIL = E_PER_W - NFULL * K
E_PER_T = N_EDGES // NS
NFULL_T = E_PER_T // K
KTAIL_T = E_PER_T - NFULL_T * K
N_PAD = 10240
ROWS_PER_TILE = N_PAD // NS
RCHUNK = 128
SEG = N_PAD // NS


def _zero_vmem_2d(buf, rows, cols):
  zeros16 = jnp.zeros((16,), jnp.float32)
  def body(i, _):
    r = i // (cols // 16)
    c = (i % (cols // 16)) * 16
    buf[r, pl.ds(c, 16)] = zeros16
    return 0
  lax.fori_loop(0, rows * (cols // 16), body, 0)


def _zero_vmem_1d(buf, n):
  zeros16 = jnp.zeros((16,), jnp.float32)
  def body(i, _):
    buf[pl.ds(i * 16, 16)] = zeros16
    return 0
  lax.fori_loop(0, n // 16, body, 0)


def _sc_degree(dst):
  mesh = plsc.VectorSubcoreMesh(core_axis_name="c", subcore_axis_name="s")

  @functools.partial(
      pl.kernel,
      out_type=jax.ShapeDtypeStruct((NC, N_PAD), jnp.float32),
      mesh=mesh,
      scratch_types=[
          pltpu.VMEM((K,), jnp.int32),
          pltpu.VMEM((KTAIL,), jnp.int32),
          pltpu.VMEM((K,), jnp.float32),
          pltpu.VMEM((SEG,), jnp.float32),
          pltpu.VMEM_SHARED((N_PAD,), jnp.float32),
      ],
  )
  def k(dst_hbm, out_hbm, didx, didx_t, ones_v, stage, shared):
    c = lax.axis_index("c")
    s = lax.axis_index("s")
    w = s * NC + c
    ebase = w * E_PER_W
    ones16 = jnp.ones((16,), jnp.float32)
    def fill(i, _):
      ones_v[pl.ds(i * 16, 16)] = ones16
      return 0
    lax.fori_loop(0, K // 16, fill, 0)
    _zero_vmem_1d(stage, SEG)
    pltpu.sync_copy(stage, shared.at[pl.ds(s * SEG, SEG)])
    plsc.subcore_barrier()
    def chunk(j, _):
      pltpu.sync_copy(dst_hbm.at[pl.ds(ebase + j * K, K)], didx)
      pltpu.sync_copy(ones_v, shared.at[didx], add=True)
      return 0
    lax.fori_loop(0, NFULL, chunk, 0)
    if KTAIL:
      pltpu.sync_copy(dst_hbm.at[pl.ds(ebase + NFULL * K, KTAIL)], didx_t)
      pltpu.sync_copy(ones_v.at[pl.ds(0, KTAIL)], shared.at[didx_t], add=True)
    plsc.subcore_barrier()
    pltpu.sync_copy(shared.at[pl.ds(s * SEG, SEG)], stage)
    pltpu.sync_copy(stage, out_hbm.at[c, pl.ds(s * SEG, SEG)])

  return k(dst)


E_HALF = N_EDGES // NC


def _sc_agg_l1(xs, src, dst):
  mesh = plsc.VectorSubcoreMesh(core_axis_name="c", subcore_axis_name="s")

  @functools.partial(
      pl.kernel,
      out_type=[
          jax.ShapeDtypeStruct((N_PAD, D_FEAT), jnp.float32),
          jax.ShapeDtypeStruct((N_PAD, D_FEAT), jnp.float32),
      ],
      mesh=mesh,
      scratch_types=[
          pltpu.VMEM((K,), jnp.int32),
          pltpu.VMEM((K,), jnp.int32),
          pltpu.VMEM((KTAIL,), jnp.int32),
          pltpu.VMEM((KTAIL,), jnp.int32),
          pltpu.VMEM((K, D_FEAT), jnp.float32),
          pltpu.VMEM((KTAIL, D_FEAT), jnp.float32),
          pltpu.VMEM((RCHUNK, D_FEAT), jnp.float32),
          pltpu.VMEM_SHARED((N_PAD, D_FEAT), jnp.float32),
          pltpu.SemaphoreType.DMA,
      ],
  )
  def k(xs_hbm, src_hbm, dst_hbm, out0_hbm, out1_hbm,
        sidx, didx, sidx_t, didx_t, rows, rows_t, buf, shared, sem):
    c = lax.axis_index("c")
    s = lax.axis_index("s")
    ebase = c * E_HALF + s * E_PER_W
    r0 = s * ROWS_PER_TILE

    _zero_vmem_2d(buf, RCHUNK, D_FEAT)
    for i in range(ROWS_PER_TILE // RCHUNK):
      pltpu.sync_copy(buf, shared.at[pl.ds(r0 + i * RCHUNK, RCHUNK), :])
    plsc.subcore_barrier()

    def chunk(j, _):
      b = ebase + j * K
      pltpu.sync_copy(src_hbm.at[pl.ds(b, K)], sidx)
      pltpu.sync_copy(dst_hbm.at[pl.ds(b, K)], didx)
      pltpu.async_copy(xs_hbm.at[sidx], rows, sem).wait()
      pltpu.sync_copy(rows, shared.at[didx], add=True)
      return 0
    lax.fori_loop(0, NFULL, chunk, 0)
    if KTAIL:
      b = ebase + NFULL * K
      pltpu.sync_copy(src_hbm.at[pl.ds(b, KTAIL)], sidx_t)
      pltpu.sync_copy(dst_hbm.at[pl.ds(b, KTAIL)], didx_t)
      pltpu.async_copy(xs_hbm.at[sidx_t], rows_t, sem).wait()
      pltpu.sync_copy(rows_t, shared.at[didx_t], add=True)
    plsc.subcore_barrier()

    def copy_out(out_hbm):
      for i in range(ROWS_PER_TILE // RCHUNK):
        rr = r0 + i * RCHUNK
        pltpu.sync_copy(shared.at[pl.ds(rr, RCHUNK), :], buf)
        pltpu.sync_copy(buf, out_hbm.at[pl.ds(rr, RCHUNK), :])

    @pl.when(c == 0)
    def _():
      copy_out(out0_hbm)

    @pl.when(c == 1)
    def _():
      copy_out(out1_hbm)

  return k(xs, src, dst)


def _sc_aggregate(xs0, xs1, src, dst, dh):
  mesh = plsc.VectorSubcoreMesh(core_axis_name="c", subcore_axis_name="s")

  @functools.partial(
      pl.kernel,
      out_type=[
          jax.ShapeDtypeStruct((N_PAD, dh), jnp.float32),
          jax.ShapeDtypeStruct((N_PAD, dh), jnp.float32),
      ],
      mesh=mesh,
      scratch_types=[
          pltpu.VMEM((K,), jnp.int32),
          pltpu.VMEM((K,), jnp.int32),
          pltpu.VMEM((KTAIL_T,), jnp.int32),
          pltpu.VMEM((KTAIL_T,), jnp.int32),
          pltpu.VMEM((K, dh), jnp.float32),
          pltpu.VMEM((KTAIL_T, dh), jnp.float32),
          pltpu.VMEM((RCHUNK, dh), jnp.float32),
          pltpu.VMEM_SHARED((N_PAD, dh), jnp.float32),
          pltpu.SemaphoreType.DMA,
      ],
  )
  def k(xs0_hbm, xs1_hbm, src_hbm, dst_hbm, out0_hbm, out1_hbm,
        sidx, didx, sidx_t, didx_t, rows, rows_t, buf, shared, sem):
    c = lax.axis_index("c")
    s = lax.axis_index("s")
    ebase = s * E_PER_T
    r0 = s * ROWS_PER_TILE

    _zero_vmem_2d(buf, RCHUNK, dh)
    for i in range(ROWS_PER_TILE // RCHUNK):
      pltpu.sync_copy(buf, shared.at[pl.ds(r0 + i * RCHUNK, RCHUNK), :])
    plsc.subcore_barrier()

    def run(xs_hbm, out_hbm):
      def chunk(j, _):
        b = ebase + j * K
        pltpu.sync_copy(src_hbm.at[pl.ds(b, K)], sidx)
        pltpu.sync_copy(dst_hbm.at[pl.ds(b, K)], didx)
        pltpu.async_copy(xs_hbm.at[sidx], rows, sem).wait()
        pltpu.sync_copy(rows, shared.at[didx], add=True)
        return 0
      lax.fori_loop(0, NFULL_T, chunk, 0)
      if KTAIL_T:
        b = ebase + NFULL_T * K
        pltpu.sync_copy(src_hbm.at[pl.ds(b, KTAIL_T)], sidx_t)
        pltpu.sync_copy(dst_hbm.at[pl.ds(b, KTAIL_T)], didx_t)
        pltpu.async_copy(xs_hbm.at[sidx_t], rows_t, sem).wait()
        pltpu.sync_copy(rows_t, shared.at[didx_t], add=True)
      plsc.subcore_barrier()
      for i in range(ROWS_PER_TILE // RCHUNK):
        rr = r0 + i * RCHUNK
        pltpu.sync_copy(shared.at[pl.ds(rr, RCHUNK), :], buf)
        pltpu.sync_copy(buf, out_hbm.at[pl.ds(rr, RCHUNK), :])

    @pl.when(c == 0)
    def _():
      run(xs0_hbm, out0_hbm)

    @pl.when(c == 1)
    def _():
      run(xs1_hbm, out1_hbm)

  return k(xs0, xs1, src, dst)


def _tc_scale(p0, p1, x):
  def body(p0_ref, p1_ref, x_ref, xs_ref, dinv_ref):
    deg = 1.0 + p0_ref[...] + p1_ref[...]
    dinv = lax.rsqrt(deg)
    dinv_ref[...] = dinv
    xs_ref[...] = x_ref[...] * dinv

  return pl.pallas_call(
      body,
      out_shape=[
          jax.ShapeDtypeStruct((N_NODES, D_FEAT), jnp.float32),
          jax.ShapeDtypeStruct((N_NODES, 1), jnp.float32),
      ],
  )(p0, p1, x)


def _tc_layer1(p0, p1, xs, dinv, W1, b1):
  nb = 5
  blk = N_NODES // nb

  def body(p0_ref, p1_ref, x_ref, d_ref, w_ref, b_ref, o0_ref, o1_ref):
    d = d_ref[...]
    a = (p0_ref[...] + p1_ref[...] + x_ref[...]) * d
    h = jnp.dot(a, w_ref[...], preferred_element_type=jnp.float32)
    h = jnp.maximum(h + b_ref[...], 0.0) * d
    o0_ref[...] = h[:, :HIDDEN // 2]
    o1_ref[...] = h[:, HIDDEN // 2:]

  rows = lambda i: (i, 0)
  return pl.pallas_call(
      body,
      grid=(nb,),
      in_specs=[
          pl.BlockSpec((blk, D_FEAT), rows),
          pl.BlockSpec((blk, D_FEAT), rows),
          pl.BlockSpec((blk, D_FEAT), rows),
          pl.BlockSpec((blk, 1), rows),
          pl.BlockSpec((D_FEAT, HIDDEN), lambda i: (0, 0)),
          pl.BlockSpec((1, HIDDEN), lambda i: (0, 0)),
      ],
      out_specs=[
          pl.BlockSpec((blk, HIDDEN // 2), rows),
          pl.BlockSpec((blk, HIDDEN // 2), rows),
      ],
      out_shape=[
          jax.ShapeDtypeStruct((N_NODES, HIDDEN // 2), jnp.float32),
          jax.ShapeDtypeStruct((N_NODES, HIDDEN // 2), jnp.float32),
      ],
  )(p0, p1, xs, dinv, W1, b1)


def _tc_head(agg0, agg1, h0, h1, dinv, W2, b2, batch3, Wfc, bfc):
  nb = 5
  blk = N_NODES // nb

  def body(a0_ref, a1_ref, h0_ref, h1_ref, d_ref, w_ref, b_ref, bat_ref,
           wfc_ref, bfc_ref, mean_ref, std_ref, pool_acc, cnt_acc):
    i = pl.program_id(0)
    d = d_ref[...]
    a_lo = (a0_ref[...] + h0_ref[...]) * d
    a_hi = (a1_ref[...] + h1_ref[...]) * d
    h = jnp.dot(a_lo, w_ref[:HIDDEN // 2, :],
                preferred_element_type=jnp.float32)
    h += jnp.dot(a_hi, w_ref[HIDDEN // 2:, :],
                 preferred_element_type=jnp.float32)
    h = jnp.maximum(h + b_ref[...], 0.0)

    bat = bat_ref[0, 0, :]
    gid = lax.broadcasted_iota(jnp.int32, (NUM_GRAPHS, blk), 0)
    onehot = (gid == bat[None, :]).astype(jnp.float32)

    @pl.when(i == 0)
    def _():
      pool_acc[...] = jnp.zeros_like(pool_acc)
      cnt_acc[...] = jnp.zeros_like(cnt_acc)

    pool_acc[...] += jnp.dot(onehot, h, preferred_element_type=jnp.float32)
    cnt_acc[...] += jnp.sum(onehot, axis=1, keepdims=True)

    @pl.when(i == nb - 1)
    def _():
      cnt = jnp.maximum(cnt_acc[...], 1.0)
      pooled = pool_acc[...] / cnt
      mls = jnp.dot(pooled, wfc_ref[...],
                    preferred_element_type=jnp.float32) + bfc_ref[...]
      mean_ref[...] = mls[:, :ACTION_DIM]
      log_std = jnp.clip(mls[:, ACTION_DIM:], -20.0, 2.0)
      std_ref[...] = jnp.exp(log_std)

  rows = lambda i: (i, 0)
  return pl.pallas_call(
      body,
      grid=(nb,),
      in_specs=[
          pl.BlockSpec((blk, HIDDEN // 2), rows),
          pl.BlockSpec((blk, HIDDEN // 2), rows),
          pl.BlockSpec((blk, HIDDEN // 2), rows),
          pl.BlockSpec((blk, HIDDEN // 2), rows),
          pl.BlockSpec((blk, 1), rows),
          pl.BlockSpec((HIDDEN, HIDDEN), lambda i: (0, 0)),
          pl.BlockSpec((1, HIDDEN), lambda i: (0, 0)),
          pl.BlockSpec((1, 1, blk), lambda i: (i, 0, 0)),
          pl.BlockSpec((HIDDEN, 2 * ACTION_DIM), lambda i: (0, 0)),
          pl.BlockSpec((1, 2 * ACTION_DIM), lambda i: (0, 0)),
      ],
      out_specs=[
          pl.BlockSpec((NUM_GRAPHS, ACTION_DIM), lambda i: (0, 0)),
          pl.BlockSpec((NUM_GRAPHS, ACTION_DIM), lambda i: (0, 0)),
      ],
      out_shape=[
          jax.ShapeDtypeStruct((NUM_GRAPHS, ACTION_DIM), jnp.float32),
          jax.ShapeDtypeStruct((NUM_GRAPHS, ACTION_DIM), jnp.float32),
      ],
      scratch_shapes=[
          pltpu.VMEM((NUM_GRAPHS, HIDDEN), jnp.float32),
          pltpu.VMEM((NUM_GRAPHS, 1), jnp.float32),
      ],
  )(agg0, agg1, h0, h1, dinv, W2, b2, batch3, Wfc, bfc)


def kernel(x, edge_index, batch, W1, b1, W2, b2, Wfc, bfc):
  src = edge_index[0]
  dst = edge_index[1]

  partial = _sc_degree(dst)
  p0 = partial[0, :N_NODES].reshape(N_NODES, 1)
  p1 = partial[1, :N_NODES].reshape(N_NODES, 1)

  xs, dinv = _tc_scale(p0, p1, x)

  agg10, agg11 = _sc_agg_l1(xs, src, dst)
  h1s0, h1s1 = _tc_layer1(agg10, agg11, xs, dinv,
                          W1, b1.reshape(1, HIDDEN))

  agg20, agg21 = _sc_aggregate(h1s0, h1s1, src, dst, HIDDEN // 2)
  mean, std = _tc_head(agg20, agg21, h1s0, h1s1, dinv,
                       W2, b2.reshape(1, HIDDEN),
                       batch.reshape(5, 1, N_NODES // 5),
                       Wfc, bfc.reshape(1, 2 * ACTION_DIM))
  return (mean, std)
